# baseline (speedup 1.0000x reference)
import functools

import jax
import jax.numpy as jnp
from jax import lax
from jax.experimental import pallas as pl
from jax.experimental.pallas import tpu as pltpu
from jax.experimental.pallas import tpu_sc as plsc

CLS_NUM = 100000
FEATURE_NUM = 64
BATCH = 16384

NC = 2
NS = 16
NW = NC * NS
B_PER_W = BATCH // NW
HIST_PAD = 100096


def _sc_body(labels_hbm, center_hbm, out_hbm,
             labv_my, labv_hist, zeros_v, ones_v, cntv, rows_vp,
             hist, sem):
    cid = lax.axis_index("c")
    sid = lax.axis_index("s")
    wid = sid * NC + cid

    for k in range(4):
        pltpu.sync_copy(
            labels_hbm.at[pl.ds((wid * 4 + k) * 128, 128)], labv_my.at[k]
        )
    cps = [
        pltpu.async_copy(
            center_hbm.at[labv_my.at[k]],
            rows_vp.at[pl.ds(k * 128, 128)],
            sem,
        )
        for k in range(4)
    ]

    for j in range(8):
        zeros_v[pl.ds(j * 16, 16)] = jnp.zeros((16,), jnp.float32)
        ones_v[pl.ds(j * 16, 16)] = jnp.ones((16,), jnp.float32)

    for k in range(8):
        pltpu.sync_copy(
            labels_hbm.at[pl.ds((sid * 8 + k) * 128, 128)], labv_hist.at[k]
        )

    for k in range(8):
        pltpu.sync_copy(zeros_v, hist.at[labv_hist.at[k]])
    plsc.subcore_barrier()
    for k in range(8):
        pltpu.sync_copy(ones_v, hist.at[labv_hist.at[k]], add=True)
    plsc.subcore_barrier()

    for k in range(4):
        pltpu.sync_copy(hist.at[labv_my.at[k]], cntv.at[pl.ds(k * 128, 128)])

    for cp in cps:
        cp.wait()

    iota = lax.iota(jnp.int32, 16)
    c64 = jnp.full((16,), FEATURE_NUM, jnp.int32)
    for g in range(B_PER_W // 16):
        cv = cntv[pl.ds(g * 16, 16)]
        plsc.store_scatter(rows_vp, [g * 16 + iota, c64], cv)

    pltpu.sync_copy(rows_vp, out_hbm.at[pl.ds(wid * B_PER_W, B_PER_W)])


_sc_gather = functools.partial(
    pl.kernel,
    mesh=plsc.VectorSubcoreMesh(core_axis_name="c", subcore_axis_name="s"),
    compiler_params=pltpu.CompilerParams(
        use_tc_tiling_on_sc=False, needs_layout_passes=False
    ),
    out_type=[
        jax.ShapeDtypeStruct((BATCH, 128), jnp.float32),
    ],
    scratch_types=[
        pltpu.VMEM((4, 128), jnp.int32),
        pltpu.VMEM((8, 128), jnp.int32),
        pltpu.VMEM((128,), jnp.float32),
        pltpu.VMEM((128,), jnp.float32),
        pltpu.VMEM((B_PER_W,), jnp.float32),
        pltpu.VMEM((B_PER_W, 128), jnp.float32),
        pltpu.VMEM_SHARED((HIST_PAD,), jnp.float32),
        pltpu.SemaphoreType.DMA,
    ],
)(_sc_body)


CPAD_ROWS = 100096


def _pt_body(ct_ref, out_ref):
    out_ref[:, :FEATURE_NUM] = ct_ref[...].T


_pad_transpose = pl.pallas_call(
    _pt_body,
    grid=(CPAD_ROWS // 128,),
    in_specs=[pl.BlockSpec((FEATURE_NUM, 128), lambda i: (0, i))],
    out_specs=pl.BlockSpec((128, 128), lambda i: (i, 0)),
    out_shape=jax.ShapeDtypeStruct((CPAD_ROWS, 128), jnp.float32),
)


TC_BLK = 2048


def _tc_body(datas_ref, slab_ref, out_ref):
    x = datas_ref[...]
    slab = slab_ref[...]
    diff = x - slab[:, :FEATURE_NUM]
    d2 = jnp.sum(diff * diff, axis=1, keepdims=True)
    cnt = slab[:, FEATURE_NUM:FEATURE_NUM + 1]
    part = jnp.sum(jnp.sqrt(d2) / cnt).reshape(1, 1)

    @pl.when(pl.program_id(0) == 0)
    def _():
        out_ref[...] = jnp.zeros_like(out_ref)

    out_ref[...] += part


_tc_tail = pl.pallas_call(
    _tc_body,
    grid=(BATCH // TC_BLK,),
    in_specs=[
        pl.BlockSpec((TC_BLK, FEATURE_NUM), lambda i: (i, 0)),
        pl.BlockSpec((TC_BLK, 128), lambda i: (i, 0)),
    ],
    out_specs=pl.BlockSpec((1, 1), lambda i: (0, 0)),
    out_shape=jax.ShapeDtypeStruct((1, 1), jnp.float32),
)


@jax.jit
def kernel(datas, labels, center):
    center128 = _pad_transpose(center.T)
    (slab,) = _sc_gather(labels.astype(jnp.int32), center128)
    out = _tc_tail(datas, slab)
    return out[0, 0]


# speedup vs baseline: 3.8073x; 3.8073x over previous
import functools

import jax
import jax.numpy as jnp
from jax import lax
from jax.experimental import pallas as pl
from jax.experimental.pallas import tpu as pltpu
from jax.experimental.pallas import tpu_sc as plsc

CLS_NUM = 100000
FEATURE_NUM = 64
BATCH = 16384

NC = 2
NS = 16
NW = NC * NS
B_PER_W = BATCH // NW
HIST_PAD = 100096


def _sc_body(labels_hbm, center_hbm, out_hbm,
             labv_my, labv_hist, zeros_v, ones_v, cntv, rows_vp,
             hist, sem):
    cid = lax.axis_index("c")
    sid = lax.axis_index("s")
    wid = sid * NC + cid

    for k in range(4):
        pltpu.sync_copy(
            labels_hbm.at[pl.ds((wid * 4 + k) * 128, 128)], labv_my.at[k]
        )
    cps = [
        pltpu.async_copy(
            center_hbm.at[labv_my.at[k]],
            rows_vp.at[pl.ds(k * 128, 128)],
            sem,
        )
        for k in range(4)
    ]

    for j in range(8):
        zeros_v[pl.ds(j * 16, 16)] = jnp.zeros((16,), jnp.float32)
        ones_v[pl.ds(j * 16, 16)] = jnp.ones((16,), jnp.float32)

    for k in range(8):
        pltpu.sync_copy(
            labels_hbm.at[pl.ds((sid * 8 + k) * 128, 128)], labv_hist.at[k]
        )

    for k in range(8):
        pltpu.sync_copy(zeros_v, hist.at[labv_hist.at[k]])
    plsc.subcore_barrier()
    for k in range(8):
        pltpu.sync_copy(ones_v, hist.at[labv_hist.at[k]], add=True)
    plsc.subcore_barrier()

    for k in range(4):
        pltpu.sync_copy(hist.at[labv_my.at[k]], cntv.at[pl.ds(k * 128, 128)])

    for cp in cps:
        cp.wait()

    iota = lax.iota(jnp.int32, 16)
    c64 = jnp.full((16,), FEATURE_NUM, jnp.int32)
    for g in range(B_PER_W // 16):
        cv = cntv[pl.ds(g * 16, 16)]
        plsc.store_scatter(rows_vp, [g * 16 + iota, c64], cv)

    pltpu.sync_copy(rows_vp, out_hbm.at[pl.ds(wid * B_PER_W, B_PER_W)])


_sc_gather = functools.partial(
    pl.kernel,
    mesh=plsc.VectorSubcoreMesh(core_axis_name="c", subcore_axis_name="s"),
    compiler_params=pltpu.CompilerParams(
        use_tc_tiling_on_sc=False, needs_layout_passes=False
    ),
    out_type=[
        jax.ShapeDtypeStruct((BATCH, 128), jnp.float32),
    ],
    scratch_types=[
        pltpu.VMEM((4, 128), jnp.int32),
        pltpu.VMEM((8, 128), jnp.int32),
        pltpu.VMEM((128,), jnp.float32),
        pltpu.VMEM((128,), jnp.float32),
        pltpu.VMEM((B_PER_W,), jnp.float32),
        pltpu.VMEM((B_PER_W, 128), jnp.float32),
        pltpu.VMEM_SHARED((HIST_PAD,), jnp.float32),
        pltpu.SemaphoreType.DMA,
    ],
)(_sc_body)


CPAD_ROWS = 100352
PT_BLK = 1024


def _pt_body(ct_ref, out_ref):
    eye = jnp.eye(FEATURE_NUM, dtype=jnp.float32)
    t = jax.lax.dot_general(
        ct_ref[...], eye, (((0,), (0,)), ((), ())),
        preferred_element_type=jnp.float32,
    )
    out_ref[:, :FEATURE_NUM] = t


_pad_transpose = pl.pallas_call(
    _pt_body,
    grid=(CPAD_ROWS // PT_BLK,),
    in_specs=[pl.BlockSpec((FEATURE_NUM, PT_BLK), lambda i: (0, i))],
    out_specs=pl.BlockSpec((PT_BLK, 128), lambda i: (i, 0)),
    out_shape=jax.ShapeDtypeStruct((CPAD_ROWS, 128), jnp.float32),
)


TC_BLK = 2048


def _tc_body(datas_ref, slab_ref, out_ref):
    x = datas_ref[...]
    slab = slab_ref[...]
    diff = x - slab[:, :FEATURE_NUM]
    d2 = jnp.sum(diff * diff, axis=1, keepdims=True)
    cnt = slab[:, FEATURE_NUM:FEATURE_NUM + 1]
    part = jnp.sum(jnp.sqrt(d2) / cnt).reshape(1, 1)

    @pl.when(pl.program_id(0) == 0)
    def _():
        out_ref[...] = jnp.zeros_like(out_ref)

    out_ref[...] += part


_tc_tail = pl.pallas_call(
    _tc_body,
    grid=(BATCH // TC_BLK,),
    in_specs=[
        pl.BlockSpec((TC_BLK, FEATURE_NUM), lambda i: (i, 0)),
        pl.BlockSpec((TC_BLK, 128), lambda i: (i, 0)),
    ],
    out_specs=pl.BlockSpec((1, 1), lambda i: (0, 0)),
    out_shape=jax.ShapeDtypeStruct((1, 1), jnp.float32),
)


@jax.jit
def kernel(datas, labels, center):
    center128 = _pad_transpose(center.T)
    (slab,) = _sc_gather(labels.astype(jnp.int32), center128)
    out = _tc_tail(datas, slab)
    return out[0, 0]


# speedup vs baseline: 6.2274x; 1.6357x over previous
import functools

import jax
import jax.numpy as jnp
from jax import lax
from jax.experimental import pallas as pl
from jax.experimental.pallas import tpu as pltpu
from jax.experimental.pallas import tpu_sc as plsc

CLS_NUM = 100000
FEATURE_NUM = 64
BATCH = 16384

TPAD = 53248
SHIFT = 47104

NC = 2
NS = 16
NW = NC * NS
B_PER_W = BATCH // NW
HIST_PAD = 100096


PK_BLK = 2048


def _pack_body(lo_ref, hi_ref, out_ref):
    eye = jnp.eye(FEATURE_NUM, dtype=jnp.float32)
    t_lo = jax.lax.dot_general(
        lo_ref[...], eye, (((0,), (0,)), ((), ())),
        preferred_element_type=jnp.float32,
    )
    t_hi = jax.lax.dot_general(
        hi_ref[...], eye, (((0,), (0,)), ((), ())),
        preferred_element_type=jnp.float32,
    )
    out_ref[:, :FEATURE_NUM] = t_lo
    out_ref[:, FEATURE_NUM:] = t_hi


_pack_table = pl.pallas_call(
    _pack_body,
    grid=(TPAD // PK_BLK,),
    in_specs=[
        pl.BlockSpec((FEATURE_NUM, PK_BLK), lambda i: (0, i)),
        pl.BlockSpec((FEATURE_NUM, PK_BLK), lambda i: (0, SHIFT // PK_BLK + i)),
    ],
    out_specs=pl.BlockSpec((PK_BLK, 128), lambda i: (i, 0)),
    out_shape=jax.ShapeDtypeStruct((TPAD, 128), jnp.float32),
)


def _sc_body(labels_hbm, table_hbm, slab_hbm, cnt_hbm,
             labv_my, labv_idx, labv_hist, zeros_v, ones_v, cntv, rows_vp,
             hist, sem):
    cid = lax.axis_index("c")
    sid = lax.axis_index("s")
    wid = sid * NC + cid

    for k in range(4):
        pltpu.sync_copy(
            labels_hbm.at[pl.ds((wid * 4 + k) * 128, 128)], labv_my.at[k]
        )

    th = jnp.full((16,), TPAD, jnp.int32)
    sh = jnp.full((16,), SHIFT, jnp.int32)
    for k in range(4):
        for j in range(8):
            v = labv_my[k, pl.ds(j * 16, 16)]
            labv_idx[k, pl.ds(j * 16, 16)] = jnp.where(v >= th, v - sh, v)

    cps = [
        pltpu.async_copy(
            table_hbm.at[labv_idx.at[k]],
            rows_vp.at[pl.ds(k * 128, 128)],
            sem,
        )
        for k in range(4)
    ]

    for j in range(8):
        zeros_v[pl.ds(j * 16, 16)] = jnp.zeros((16,), jnp.float32)
        ones_v[pl.ds(j * 16, 16)] = jnp.ones((16,), jnp.float32)

    for k in range(8):
        pltpu.sync_copy(
            labels_hbm.at[pl.ds((sid * 8 + k) * 128, 128)], labv_hist.at[k]
        )

    for k in range(8):
        pltpu.sync_copy(zeros_v, hist.at[labv_hist.at[k]])
    plsc.subcore_barrier()
    for k in range(8):
        pltpu.sync_copy(ones_v, hist.at[labv_hist.at[k]], add=True)
    plsc.subcore_barrier()

    for k in range(4):
        pltpu.sync_copy(hist.at[labv_my.at[k]], cntv.at[pl.ds(k * 128, 128)])
    pltpu.sync_copy(cntv, cnt_hbm.at[pl.ds(wid * B_PER_W, B_PER_W)])

    for cp in cps:
        cp.wait()
    pltpu.sync_copy(rows_vp, slab_hbm.at[pl.ds(wid * B_PER_W, B_PER_W)])


_sc_gather = functools.partial(
    pl.kernel,
    mesh=plsc.VectorSubcoreMesh(core_axis_name="c", subcore_axis_name="s"),
    compiler_params=pltpu.CompilerParams(
        use_tc_tiling_on_sc=False, needs_layout_passes=False
    ),
    out_type=[
        jax.ShapeDtypeStruct((BATCH, 128), jnp.float32),
        jax.ShapeDtypeStruct((BATCH,), jnp.float32),
    ],
    scratch_types=[
        pltpu.VMEM((4, 128), jnp.int32),
        pltpu.VMEM((4, 128), jnp.int32),
        pltpu.VMEM((8, 128), jnp.int32),
        pltpu.VMEM((128,), jnp.float32),
        pltpu.VMEM((128,), jnp.float32),
        pltpu.VMEM((B_PER_W,), jnp.float32),
        pltpu.VMEM((B_PER_W, 128), jnp.float32),
        pltpu.VMEM_SHARED((HIST_PAD,), jnp.float32),
        pltpu.SemaphoreType.DMA,
    ],
)(_sc_body)


TC_BLK = 2048


def _tc_body(xT_ref, slab_ref, lab_ref, cnt_ref, out_ref):
    xT = xT_ref[...]
    slab = slab_ref[...]
    eye = jnp.eye(128, dtype=jnp.float32)
    st = jax.lax.dot_general(
        eye, slab, (((1,), (1,)), ((), ())),
        preferred_element_type=jnp.float32,
    )
    diff_lo = xT - st[:FEATURE_NUM, :]
    diff_hi = xT - st[FEATURE_NUM:, :]
    d2_lo = jnp.sum(diff_lo * diff_lo, axis=0)
    d2_hi = jnp.sum(diff_hi * diff_hi, axis=0)
    sel = lab_ref[...] >= TPAD
    d2 = jnp.where(sel, d2_hi, d2_lo)
    part = jnp.sum(jnp.sqrt(d2) / cnt_ref[...]).reshape(1, 1)

    @pl.when(pl.program_id(0) == 0)
    def _():
        out_ref[...] = jnp.zeros_like(out_ref)

    out_ref[...] += part


_tc_tail = pl.pallas_call(
    _tc_body,
    grid=(BATCH // TC_BLK,),
    in_specs=[
        pl.BlockSpec((FEATURE_NUM, TC_BLK), lambda i: (0, i)),
        pl.BlockSpec((TC_BLK, 128), lambda i: (i, 0)),
        pl.BlockSpec((TC_BLK,), lambda i: (i,)),
        pl.BlockSpec((TC_BLK,), lambda i: (i,)),
    ],
    out_specs=pl.BlockSpec((1, 1), lambda i: (0, 0)),
    out_shape=jax.ShapeDtypeStruct((1, 1), jnp.float32),
)


@jax.jit
def kernel(datas, labels, center):
    labels = labels.astype(jnp.int32)
    centerT = center.T
    table = _pack_table(centerT, centerT)
    slab, cnt = _sc_gather(labels, table)
    out = _tc_tail(datas.T, slab, labels, cnt)
    return out[0, 0]
